# Initial kernel scaffold; baseline (speedup 1.0000x reference)
import jax
import jax.numpy as jnp
from jax.experimental import pallas as pl


def kernel(x, table):
    raise NotImplementedError("write your pallas kernel here")


# speedup vs baseline: 1.2847x; 1.2847x over previous
import functools

import jax
import jax.numpy as jnp
from jax import lax
from jax.experimental import pallas as pl
from jax.experimental.pallas import tpu as pltpu
from jax.experimental.pallas import tpu_sc as plsc

_D = 128
_B = 4096 * 50
_NC = 2
_NS = 16
_NW = _NC * _NS
_BPW = _B // _NW
_CH = 128
_NCH = _BPW // _CH
_NBUF = 5
_NGRP = _NCH // _NBUF

_mesh = plsc.VectorSubcoreMesh(core_axis_name="c", subcore_axis_name="s")


@functools.partial(
    pl.kernel,
    mesh=_mesh,
    out_type=jax.ShapeDtypeStruct((_B, _D), jnp.float32),
    scratch_types=[
        pltpu.VMEM((_NCH, _CH), jnp.int32),
        pltpu.VMEM((_NBUF, _CH, _D), jnp.float32),
        pltpu.SemaphoreType.DMA,
        pltpu.SemaphoreType.DMA,
    ],
)
def _embed_lookup(idx_hbm, table_hbm, out_hbm, idx_v, rows_v, gsem, ssem):
    wid = lax.axis_index("s") * _NC + lax.axis_index("c")
    base = wid * _BPW
    pltpu.sync_copy(idx_hbm.at[wid], idx_v)

    def gather(j, b):
        return pltpu.make_async_copy(
            table_hbm.at[idx_v.at[j]], rows_v.at[b], gsem)

    def scatter(j, b):
        return pltpu.make_async_copy(
            rows_v.at[b], out_hbm.at[pl.ds(base + j * _CH, _CH)], ssem)

    for b in range(_NBUF):
        gather(b, b).start()

    @pl.loop(0, _NGRP - 1)
    def _grp(g):
        j0 = g * _NBUF
        for b in range(_NBUF):
            gather(j0 + b, b).wait()
            scatter(j0 + b, b).start()
        for b in range(_NBUF):
            scatter(j0 + b, b).wait()
            gather(j0 + _NBUF + b, b).start()

    j0 = (_NGRP - 1) * _NBUF
    for b in range(_NBUF):
        gather(j0 + b, b).wait()
        scatter(j0 + b, b).start()
    for b in range(_NBUF):
        scatter(j0 + b, b).wait()


def kernel(x, table):
    flat = _embed_lookup(x.reshape(_NW, _NCH, _CH), table)
    return flat.reshape(x.shape[0], x.shape[1], _D)


# speedup vs baseline: 2.2923x; 1.7843x over previous
import functools

import jax
import jax.numpy as jnp
from jax import lax
from jax.experimental import pallas as pl
from jax.experimental.pallas import tpu as pltpu
from jax.experimental.pallas import tpu_sc as plsc

_D = 128
_BATCH = 4096
_SEQ = 50
_NC = 2
_NS = 16
_NW = _NC * _NS
_RPW = _BATCH // _NW
_NBUF = 8
_NGRP = _RPW // _NBUF

_mesh = plsc.VectorSubcoreMesh(core_axis_name="c", subcore_axis_name="s")


@functools.partial(
    pl.kernel,
    mesh=_mesh,
    out_type=jax.ShapeDtypeStruct((_BATCH, _SEQ, _D), jnp.float32),
    scratch_types=[
        pltpu.VMEM((_RPW, _SEQ), jnp.int32),
        pltpu.VMEM((_NBUF, _SEQ, _D), jnp.float32),
        pltpu.SemaphoreType.DMA,
        pltpu.SemaphoreType.DMA,
    ],
)
def _embed_lookup(idx_hbm, table_hbm, out_hbm, idx_v, rows_v, gsem, ssem):
    wid = lax.axis_index("s") * _NC + lax.axis_index("c")
    base = wid * _RPW
    pltpu.sync_copy(idx_hbm.at[wid], idx_v)

    def gather(j, b):
        return pltpu.make_async_copy(
            table_hbm.at[idx_v.at[j]], rows_v.at[b], gsem)

    def scatter(j, b):
        return pltpu.make_async_copy(
            rows_v.at[b], out_hbm.at[base + j], ssem)

    for b in range(_NBUF):
        gather(b, b).start()

    @pl.loop(0, _NGRP - 1)
    def _grp(g):
        j0 = g * _NBUF
        for b in range(_NBUF):
            gather(j0 + b, b).wait()
            scatter(j0 + b, b).start()
        for b in range(_NBUF):
            scatter(j0 + b, b).wait()
            gather(j0 + _NBUF + b, b).start()

    j0 = (_NGRP - 1) * _NBUF
    for b in range(_NBUF):
        gather(j0 + b, b).wait()
        scatter(j0 + b, b).start()
    for b in range(_NBUF):
        scatter(j0 + b, b).wait()


def kernel(x, table):
    return _embed_lookup(x.reshape(_NW, _RPW, _SEQ), table)
